# baseline (speedup 1.0000x reference)
import functools

import jax
import jax.numpy as jnp
from jax import lax
from jax.experimental import pallas as pl
from jax.experimental.pallas import tpu as pltpu
from jax.experimental.pallas import tpu_sc as plsc

N = 10000
E = 320000
D = 128
E2 = 327680
N2 = 10240
BN = 2000
BE = 2048

_NC, _NS = 2, 16
_NW = _NC * _NS
_RW = E2 // _NW
_J = 128
_GJ = 80
_GB = 4
_GBR = _GJ * _GB
_GNBI = 4
_GCH = _GBR * _GNBI
_GNCH = _RW // _GCH
_SKJ = 8
_SSTG = _SKJ * _J
_SRW = E2 // _NS
_SNST = _SRW // _SSTG
_HSEG = N2 // _NC
_ACCR = _HSEG + _J

_F32 = jnp.float32


def _dot(a, b):
    return jnp.dot(a, b, preferred_element_type=_F32)


def _silu(x):
    return x * jax.nn.sigmoid(x)


def _full(shape):
    return pl.BlockSpec(shape, lambda i: (0, 0))


def _rows(block):
    return pl.BlockSpec(block, lambda i: (i, 0))


def _enc_node_body(nb, W0, b0, W1, b1, W2, b2, x_out):
    h = _silu(_dot(nb[...], W0[...]) + b0[...])
    h = _silu(_dot(h, W1[...]) + b1[...])
    x_out[...] = _dot(h, W2[...]) + b2[...]


def _enc_node(n, w):
    (W0, b0), (W1, b1), (W2, b2) = w
    grid = N // BN
    f = pl.pallas_call(
        _enc_node_body,
        grid=(grid,),
        in_specs=[_rows((BN, D))] + [_full(a.shape) for a in
                  (W0, b0, W1, b1, W2, b2)],
        out_specs=_rows((BN, D)),
        out_shape=jax.ShapeDtypeStruct((N, D), _F32),
    )
    return f(n, W0, b0, W1, b1, W2, b2)


def _edge_pass1_body(qs, qd, gs, gd, V1p, c1, V2, c2, V3, c3,
                     W1, b1, W2e, b2e, W3e, b3e, ea_out, er_out):
    d = qs[...] - qd[...]
    nrm = jnp.sqrt(jnp.sum(d * d, axis=1, keepdims=True))
    u8 = jnp.concatenate([d[:, 0:3], nrm, d[:, 4:8]], axis=1)
    h = _silu(_dot(u8, V1p[...]) + c1[...])
    h = _silu(_dot(h, V2[...]) + c2[...])
    ea = _dot(h, V3[...]) + c3[...]
    cat = jnp.concatenate([ea, gs[...], gd[...]], axis=1)
    z = _silu(_dot(cat, W1[...]) + b1[...])
    z = _silu(_dot(z, W2e[...]) + b2e[...])
    er = _dot(z, W3e[...]) + b3e[...]
    ea_out[...] = ea + er
    er_out[...] = er


def _edge_pass1(qs, qd, gs, gd, enc_w, mlp_w):
    V1p, c1, V2, c2, V3, c3 = enc_w
    W1, b1, W2e, b2e, W3e, b3e = mlp_w
    grid = E2 // BE
    f = pl.pallas_call(
        _edge_pass1_body,
        grid=(grid,),
        in_specs=[_rows((BE, D))] * 4 +
                 [_full(a.shape) for a in
                  (V1p, c1, V2, c2, V3, c3, W1, b1, W2e, b2e, W3e, b3e)],
        out_specs=[_rows((BE, D))] * 2,
        out_shape=[jax.ShapeDtypeStruct((E2, D), _F32)] * 2,
    )
    return f(qs, qd, gs, gd, V1p, c1, V2, c2, V3, c3,
             W1, b1, W2e, b2e, W3e, b3e)


def _edge_pass2_body(ea, gs, gd, W1, b1, W2e, b2e, W3e, b3e, er_out):
    cat = jnp.concatenate([ea[...], gs[...], gd[...]], axis=1)
    z = _silu(_dot(cat, W1[...]) + b1[...])
    z = _silu(_dot(z, W2e[...]) + b2e[...])
    er_out[...] = _dot(z, W3e[...]) + b3e[...]


def _edge_pass2(ea, gs, gd, mlp_w):
    W1, b1, W2e, b2e, W3e, b3e = mlp_w
    grid = E2 // BE
    f = pl.pallas_call(
        _edge_pass2_body,
        grid=(grid,),
        in_specs=[_rows((BE, D))] * 3 +
                 [_full(a.shape) for a in (W1, b1, W2e, b2e, W3e, b3e)],
        out_specs=_rows((BE, D)),
        out_shape=jax.ShapeDtypeStruct((E2, D), _F32),
    )
    return f(ea, gs, gd, W1, b1, W2e, b2e, W3e, b3e)


def _node_pass_body(x, a0, Wn1, bn1, Wn2, bn2, Wn3, bn3, x_out):
    cat = jnp.concatenate([x[...], a0[...]], axis=1)
    z = _silu(_dot(cat, Wn1[...]) + bn1[...])
    z = _silu(_dot(z, Wn2[...]) + bn2[...])
    x_out[...] = x[...] + _dot(z, Wn3[...]) + bn3[...]


def _node_pass(x, a0, node_w):
    Wn1, bn1, Wn2, bn2, Wn3, bn3 = node_w
    grid = N // BN
    f = pl.pallas_call(
        _node_pass_body,
        grid=(grid,),
        in_specs=[_rows((BN, D))] * 2 +
                 [_full(a.shape) for a in
                  (Wn1, bn1, Wn2, bn2, Wn3, bn3)],
        out_specs=_rows((BN, D)),
        out_shape=jax.ShapeDtypeStruct((N, D), _F32),
    )
    return f(x, a0, Wn1, bn1, Wn2, bn2, Wn3, bn3)


def _node_dec_body(x, a0, Wn1, bn1, Wn2, bn2, Wn3, bn3,
                   D1, e1, D2, e2, D3, e3, out):
    cat = jnp.concatenate([x[...], a0[...]], axis=1)
    z = _silu(_dot(cat, Wn1[...]) + bn1[...])
    z = _silu(_dot(z, Wn2[...]) + bn2[...])
    xn = x[...] + _dot(z, Wn3[...]) + bn3[...]
    h = _silu(_dot(xn, D1[...]) + e1[...])
    h = _silu(_dot(h, D2[...]) + e2[...])
    out[...] = _dot(h, D3[...]) + e3[...]


def _node_dec(x, a0, node_w, dec_w):
    Wn1, bn1, Wn2, bn2, Wn3, bn3 = node_w
    D1, e1, D2, e2, D3, e3 = dec_w
    grid = N // BN
    f = pl.pallas_call(
        _node_dec_body,
        grid=(grid,),
        in_specs=[_rows((BN, D))] * 2 +
                 [_full(a.shape) for a in
                  (Wn1, bn1, Wn2, bn2, Wn3, bn3, D1, e1, D2, e2, D3, e3)],
        out_specs=_rows((BN, D)),
        out_shape=jax.ShapeDtypeStruct((N, D), _F32),
    )
    return f(x, a0, Wn1, bn1, Wn2, bn2, Wn3, bn3,
             D1, e1, D2, e2, D3, e3)


def _sc_mesh():
    return plsc.VectorSubcoreMesh(core_axis_name="c", subcore_axis_name="s")


def _worker_id():
    return (1 - lax.axis_index("c")) * _NS + lax.axis_index("s")


def _gather_one(table, idx1d, out, idx_v, rows2, semg, semw, ebase):
    def chunk(g, carry):
        cbase = ebase + g * _GCH
        pltpu.sync_copy(idx1d.at[pl.ds(cbase, _GCH)], idx_v)
        for k in range(_GNBI):
            t = g * _GNBI + k
            buf = rows2.at[k % 2]

            @pl.when(t >= 2)
            def _drain():
                pltpu.make_async_copy(out.at[pl.ds(ebase, _GBR)], buf,
                                      semw).wait()
            cps = [pltpu.async_copy(
                       table.at[idx_v.at[pl.ds((k * _GB + j) * _GJ, _GJ)]],
                       buf.at[pl.ds(j * _GJ, _GJ)], semg)
                   for j in range(_GB)]
            for c in cps:
                c.wait()
            pltpu.async_copy(buf, out.at[pl.ds(cbase + k * _GBR, _GBR)], semw)
        return carry
    lax.fori_loop(0, _GNCH, chunk, 0)
    for k in range(2):
        pltpu.make_async_copy(out.at[pl.ds(ebase, _GBR)], rows2.at[k],
                              semw).wait()


def _gather4_call(q0p, xb, xc, src, dest):
    out_type = [jax.ShapeDtypeStruct((E2, D), _F32)] * 4
    scratch = [pltpu.VMEM((_GCH,), jnp.int32),
               pltpu.VMEM((2, _GBR, D), _F32),
               pltpu.SemaphoreType.DMA,
               pltpu.SemaphoreType.DMA]

    @functools.partial(pl.kernel, mesh=_sc_mesh(), out_type=out_type,
                       scratch_types=scratch)
    def k(q_h, xb_h, xc_h, srcI, destI, qs_o, qd_o, gs_o, gd_o,
          idx_v, rows2, semg, semw):
        ebase = _worker_id() * _RW
        _gather_one(q_h, srcI, qs_o, idx_v, rows2, semg, semw, ebase)
        _gather_one(q_h, destI, qd_o, idx_v, rows2, semg, semw, ebase)
        _gather_one(xb_h, srcI, gs_o, idx_v, rows2, semg, semw, ebase)
        _gather_one(xc_h, destI, gd_o, idx_v, rows2, semg, semw, ebase)

    return k(q0p, xb, xc, src, dest)


def _gather2_call(xb, xc, src, dest):
    out_type = [jax.ShapeDtypeStruct((E2, D), _F32)] * 2
    scratch = [pltpu.VMEM((_GCH,), jnp.int32),
               pltpu.VMEM((2, _GBR, D), _F32),
               pltpu.SemaphoreType.DMA,
               pltpu.SemaphoreType.DMA]

    @functools.partial(pl.kernel, mesh=_sc_mesh(), out_type=out_type,
                       scratch_types=scratch)
    def k(xb_h, xc_h, srcI, destI, gs_o, gd_o, idx_v, rows2, semg, semw):
        ebase = _worker_id() * _RW
        _gather_one(xb_h, srcI, gs_o, idx_v, rows2, semg, semw, ebase)
        _gather_one(xc_h, destI, gd_o, idx_v, rows2, semg, semw, ebase)

    return k(xb, xc, src, dest)


def _scatter_call(er, dest2d, zrows):
    out_type = jax.ShapeDtypeStruct((N2, D), _F32)
    scratch = [pltpu.VMEM((_SKJ, _J), jnp.int32),
               pltpu.VMEM((_SKJ, _J), jnp.int32),
               pltpu.VMEM((_SSTG // 2, D), _F32),
               pltpu.VMEM((_J, D), _F32),
               pltpu.VMEM_SHARED((_ACCR, D), _F32),
               pltpu.SemaphoreType.DMA]

    @functools.partial(pl.kernel, mesh=_sc_mesh(), out_type=out_type,
                       scratch_types=scratch)
    def k(er_h, destI, z_h, out, idx_v, lidx_v, rows_v, zb, acc, sem):
        cid = lax.axis_index("c")
        sid = lax.axis_index("s")
        ibase = sid * (_SRW // _J)
        ebase = sid * _SRW
        segbase = cid * _HSEG
        pltpu.sync_copy(z_h, zb)
        z0 = sid * (_ACCR // _NS)
        for off, ln in ((0, _J), (_J, _J), (2 * _J, _ACCR // _NS - 2 * _J)):
            pltpu.sync_copy(zb.at[pl.ds(0, ln)], acc.at[pl.ds(z0 + off, ln)])
        plsc.subcore_barrier()

        def body(g, carry):
            pltpu.sync_copy(destI.at[pl.ds(ibase + g * _SKJ, _SKJ)], idx_v)
            for r in range(_SKJ):
                for q in range(_J // 16):
                    v = idx_v[r, pl.ds(q * 16, 16)]
                    l = v - segbase
                    ok = (l >= 0) & (l < _HSEG)
                    dump = _HSEG + (v & (_J - 1))
                    lidx_v[r, pl.ds(q * 16, 16)] = jnp.where(ok, l, dump)
            for h in range(2):
                pltpu.sync_copy(
                    er_h.at[pl.ds(ebase + g * _SSTG + h * (_SSTG // 2),
                                  _SSTG // 2)], rows_v)
                for j in range(_SKJ // 2):
                    pltpu.sync_copy(rows_v.at[pl.ds(j * _J, _J)],
                                    acc.at[lidx_v.at[h * (_SKJ // 2) + j]],
                                    add=True)
            return carry
        lax.fori_loop(0, _SNST, body, 0)
        plsc.subcore_barrier()

        w0 = sid * (_HSEG // _NS)
        for off, ln in ((0, _J), (_J, _J), (2 * _J, _HSEG // _NS - 2 * _J)):
            pltpu.sync_copy(acc.at[pl.ds(w0 + off, ln)], zb.at[pl.ds(0, ln)])
            pltpu.sync_copy(zb.at[pl.ds(0, ln)],
                            out.at[pl.ds(segbase + w0 + off, ln)])

    return k(er, dest2d, zrows)


def _row(b):
    return b.reshape(1, D)


def kernel(n, edge_index, q_0, params):
    src = edge_index[0]
    dest = edge_index[1]

    enc_node_w = [(W, _row(b)) for W, b in params["enc_node"]]

    (V1, c1), (V2, c2), (V3, c3) = params["enc_edge"]
    V1p = jnp.zeros((8, D), _F32).at[0:4].set(V1)
    enc_edge_w = (V1p, _row(c1), V2, _row(c2), V3, _row(c3))

    edge_w = []
    for (W1, b1), (W2, b2), (W3, b3) in params["edge_mlps"]:
        edge_w.append((W1, _row(b1), W2, _row(b2), W3, _row(b3)))
    node_w = []
    for (W1, b1), (W2, b2), (W3, b3) in params["node_mlps"]:
        node_w.append((W1, _row(b1), W2, _row(b2), W3, _row(b3)))

    (D1, e1), (D2, e2), (D3, e3) = params["dec"]
    D3p = jnp.zeros((D, D), _F32).at[:, 0:3].set(D3)
    e3p = jnp.zeros((1, D), _F32).at[:, 0:3].set(e3.reshape(1, 3))
    dec_w = (D1, _row(e1), D2, _row(e2), D3p, e3p)

    q0p = jnp.zeros((N, D), _F32).at[:, 0:3].set(q_0)

    pad = E2 - E
    src_p = jnp.concatenate([src, jnp.zeros((pad,), jnp.int32)])
    dest_p = jnp.concatenate([dest, jnp.zeros((pad,), jnp.int32)])
    dscat = jnp.concatenate([dest, jnp.full((pad,), N, jnp.int32)])
    dest2d = dscat.reshape(E2 // _J, _J)
    zrows = jnp.zeros((_J, D), _F32)

    x0 = _enc_node(n, enc_node_w)

    qs, qd, gs, gd = _gather4_call(q0p, x0, x0, src_p, dest_p)

    ea1, er1 = _edge_pass1(qs, qd, gs, gd, enc_edge_w, edge_w[0])

    agg = _scatter_call(er1, dest2d, zrows)
    x1 = _node_pass(x0, agg, node_w[0])

    gs2, gd2 = _gather2_call(x1, x1, src_p, dest_p)
    er2 = _edge_pass2(ea1, gs2, gd2, edge_w[1])

    agg = _scatter_call(er2, dest2d, zrows)
    out = _node_dec(x1, agg, node_w[1], dec_w)
    return out[:, 0:3]


# speedup vs baseline: 1.0878x; 1.0878x over previous
import functools

import jax
import jax.numpy as jnp
from jax import lax
from jax.experimental import pallas as pl
from jax.experimental.pallas import tpu as pltpu
from jax.experimental.pallas import tpu_sc as plsc

N = 10000
E = 320000
D = 128
E2 = 327680
N2 = 10240
BN = 2000
BE = 2048

_NC, _NS = 2, 16
_NW = _NC * _NS
_RW = E2 // _NW
_J = 128
_GJ = 80
_GB = 4
_GBR = _GJ * _GB
_GNBI = 4
_GCH = _GBR * _GNBI
_GNCH = _RW // _GCH
_SKJ = 8
_SSTG = _SKJ * _J
_SRW = E2 // _NS
_SNST = _SRW // _SSTG
_HSEG = N2 // _NC
_ACCR = _HSEG + _J

_F32 = jnp.float32


def _dot(a, b):
    return jnp.dot(a, b, preferred_element_type=_F32)


def _silu(x):
    return x * jax.nn.sigmoid(x)


def _full(shape):
    return pl.BlockSpec(shape, lambda i: (0, 0))


def _rows(block):
    return pl.BlockSpec(block, lambda i: (i, 0))


def _enc_node_body(nb, W0, b0, W1, b1, W2, b2, x_out):
    h = _silu(_dot(nb[...], W0[...]) + b0[...])
    h = _silu(_dot(h, W1[...]) + b1[...])
    x_out[...] = _dot(h, W2[...]) + b2[...]


def _enc_node(n, w):
    (W0, b0), (W1, b1), (W2, b2) = w
    grid = N // BN
    f = pl.pallas_call(
        _enc_node_body,
        grid=(grid,),
        in_specs=[_rows((BN, D))] + [_full(a.shape) for a in
                  (W0, b0, W1, b1, W2, b2)],
        out_specs=_rows((BN, D)),
        out_shape=jax.ShapeDtypeStruct((N, D), _F32),
    )
    return f(n, W0, b0, W1, b1, W2, b2)


def _edge_pass1_body(qs, qd, gs, gd, V1p, c1, V2, c2, V3, c3,
                     W1, b1, W2e, b2e, W3e, b3e, ea_out, er_out):
    d = qs[...] - qd[...]
    nrm = jnp.sqrt(jnp.sum(d * d, axis=1, keepdims=True))
    u8 = jnp.concatenate([d[:, 0:3], nrm, d[:, 4:8]], axis=1)
    h = _silu(_dot(u8, V1p[...]) + c1[...])
    h = _silu(_dot(h, V2[...]) + c2[...])
    ea = _dot(h, V3[...]) + c3[...]
    cat = jnp.concatenate([ea, gs[...], gd[...]], axis=1)
    z = _silu(_dot(cat, W1[...]) + b1[...])
    z = _silu(_dot(z, W2e[...]) + b2e[...])
    er = _dot(z, W3e[...]) + b3e[...]
    ea_out[...] = ea + er
    er_out[...] = er


def _edge_pass1(qs, qd, gs, gd, enc_w, mlp_w):
    V1p, c1, V2, c2, V3, c3 = enc_w
    W1, b1, W2e, b2e, W3e, b3e = mlp_w
    grid = E2 // BE
    f = pl.pallas_call(
        _edge_pass1_body,
        grid=(grid,),
        in_specs=[_rows((BE, D))] * 4 +
                 [_full(a.shape) for a in
                  (V1p, c1, V2, c2, V3, c3, W1, b1, W2e, b2e, W3e, b3e)],
        out_specs=[_rows((BE, D))] * 2,
        out_shape=[jax.ShapeDtypeStruct((E2, D), _F32)] * 2,
    )
    return f(qs, qd, gs, gd, V1p, c1, V2, c2, V3, c3,
             W1, b1, W2e, b2e, W3e, b3e)


def _edge_pass2_body(ea, gs, gd, W1, b1, W2e, b2e, W3e, b3e, er_out):
    cat = jnp.concatenate([ea[...], gs[...], gd[...]], axis=1)
    z = _silu(_dot(cat, W1[...]) + b1[...])
    z = _silu(_dot(z, W2e[...]) + b2e[...])
    er_out[...] = _dot(z, W3e[...]) + b3e[...]


def _edge_pass2(ea, gs, gd, mlp_w):
    W1, b1, W2e, b2e, W3e, b3e = mlp_w
    grid = E2 // BE
    f = pl.pallas_call(
        _edge_pass2_body,
        grid=(grid,),
        in_specs=[_rows((BE, D))] * 3 +
                 [_full(a.shape) for a in (W1, b1, W2e, b2e, W3e, b3e)],
        out_specs=_rows((BE, D)),
        out_shape=jax.ShapeDtypeStruct((E2, D), _F32),
    )
    return f(ea, gs, gd, W1, b1, W2e, b2e, W3e, b3e)


def _node_pass_body(x, a0, Wn1, bn1, Wn2, bn2, Wn3, bn3, x_out):
    cat = jnp.concatenate([x[...], a0[...]], axis=1)
    z = _silu(_dot(cat, Wn1[...]) + bn1[...])
    z = _silu(_dot(z, Wn2[...]) + bn2[...])
    x_out[...] = x[...] + _dot(z, Wn3[...]) + bn3[...]


def _node_pass(x, a0, node_w):
    Wn1, bn1, Wn2, bn2, Wn3, bn3 = node_w
    grid = N // BN
    f = pl.pallas_call(
        _node_pass_body,
        grid=(grid,),
        in_specs=[_rows((BN, D))] * 2 +
                 [_full(a.shape) for a in
                  (Wn1, bn1, Wn2, bn2, Wn3, bn3)],
        out_specs=_rows((BN, D)),
        out_shape=jax.ShapeDtypeStruct((N, D), _F32),
    )
    return f(x, a0, Wn1, bn1, Wn2, bn2, Wn3, bn3)


def _node_dec_body(x, a0, Wn1, bn1, Wn2, bn2, Wn3, bn3,
                   D1, e1, D2, e2, D3, e3, out):
    cat = jnp.concatenate([x[...], a0[...]], axis=1)
    z = _silu(_dot(cat, Wn1[...]) + bn1[...])
    z = _silu(_dot(z, Wn2[...]) + bn2[...])
    xn = x[...] + _dot(z, Wn3[...]) + bn3[...]
    h = _silu(_dot(xn, D1[...]) + e1[...])
    h = _silu(_dot(h, D2[...]) + e2[...])
    out[...] = _dot(h, D3[...]) + e3[...]


def _node_dec(x, a0, node_w, dec_w):
    Wn1, bn1, Wn2, bn2, Wn3, bn3 = node_w
    D1, e1, D2, e2, D3, e3 = dec_w
    grid = N // BN
    f = pl.pallas_call(
        _node_dec_body,
        grid=(grid,),
        in_specs=[_rows((BN, D))] * 2 +
                 [_full(a.shape) for a in
                  (Wn1, bn1, Wn2, bn2, Wn3, bn3, D1, e1, D2, e2, D3, e3)],
        out_specs=_rows((BN, D)),
        out_shape=jax.ShapeDtypeStruct((N, D), _F32),
    )
    return f(x, a0, Wn1, bn1, Wn2, bn2, Wn3, bn3,
             D1, e1, D2, e2, D3, e3)


def _sc_mesh():
    return plsc.VectorSubcoreMesh(core_axis_name="c", subcore_axis_name="s")


def _worker_id():
    return lax.axis_index("c") * _NS + lax.axis_index("s")


def _gather_one(table, idx3, out3, idx_v, rows2, semg, semw, wid):
    def chunk(g, carry):
        ci = wid + g * _NW
        ov = out3.at[ci]
        pltpu.sync_copy(idx3.at[ci], idx_v)
        for k in range(_GNBI):
            t = g * _GNBI + k
            buf = rows2.at[k % 2]

            @pl.when(t >= 2)
            def _drain():
                pltpu.make_async_copy(ov.at[pl.ds(0, _GBR)], buf,
                                      semw).wait()
            cps = [pltpu.async_copy(
                       table.at[idx_v.at[pl.ds((k * _GB + j) * _GJ, _GJ)]],
                       buf.at[pl.ds(j * _GJ, _GJ)], semg)
                   for j in range(_GB)]
            for c in cps:
                c.wait()
            pltpu.async_copy(buf, ov.at[pl.ds(k * _GBR, _GBR)], semw)
        return carry
    lax.fori_loop(0, _GNCH, chunk, 0)
    for k in range(2):
        pltpu.make_async_copy(out3.at[0].at[pl.ds(0, _GBR)], rows2.at[k],
                              semw).wait()


def _gather4_call(q0p, xb, xc, src, dest):
    out_type = [jax.ShapeDtypeStruct((E2 // _GCH, _GCH, D), _F32)] * 4
    scratch = [pltpu.VMEM((_GCH,), jnp.int32),
               pltpu.VMEM((2, _GBR, D), _F32),
               pltpu.SemaphoreType.DMA,
               pltpu.SemaphoreType.DMA]

    @functools.partial(pl.kernel, mesh=_sc_mesh(), out_type=out_type,
                       scratch_types=scratch)
    def k(q_h, xb_h, xc_h, srcI, destI, qs_o, qd_o, gs_o, gd_o,
          idx_v, rows2, semg, semw):
        wid = _worker_id()
        _gather_one(q_h, srcI, qs_o, idx_v, rows2, semg, semw, wid)
        _gather_one(q_h, destI, qd_o, idx_v, rows2, semg, semw, wid)
        _gather_one(xb_h, srcI, gs_o, idx_v, rows2, semg, semw, wid)
        _gather_one(xc_h, destI, gd_o, idx_v, rows2, semg, semw, wid)

    r = k(q0p, xb, xc, src, dest)
    return [a.reshape(E2, D) for a in r]


def _gather2_call(xb, xc, src, dest):
    out_type = [jax.ShapeDtypeStruct((E2 // _GCH, _GCH, D), _F32)] * 2
    scratch = [pltpu.VMEM((_GCH,), jnp.int32),
               pltpu.VMEM((2, _GBR, D), _F32),
               pltpu.SemaphoreType.DMA,
               pltpu.SemaphoreType.DMA]

    @functools.partial(pl.kernel, mesh=_sc_mesh(), out_type=out_type,
                       scratch_types=scratch)
    def k(xb_h, xc_h, srcI, destI, gs_o, gd_o, idx_v, rows2, semg, semw):
        wid = _worker_id()
        _gather_one(xb_h, srcI, gs_o, idx_v, rows2, semg, semw, wid)
        _gather_one(xc_h, destI, gd_o, idx_v, rows2, semg, semw, wid)

    r = k(xb, xc, src, dest)
    return [a.reshape(E2, D) for a in r]


def _scatter_call(er, dest2d, zrows):
    out_type = jax.ShapeDtypeStruct((N2, D), _F32)
    scratch = [pltpu.VMEM((_SKJ, _J), jnp.int32),
               pltpu.VMEM((_SKJ, _J), jnp.int32),
               pltpu.VMEM((_SSTG // 2, D), _F32),
               pltpu.VMEM((_J, D), _F32),
               pltpu.VMEM_SHARED((_ACCR, D), _F32),
               pltpu.SemaphoreType.DMA]

    @functools.partial(pl.kernel, mesh=_sc_mesh(), out_type=out_type,
                       scratch_types=scratch)
    def k(er_h, destI, z_h, out, idx_v, lidx_v, rows_v, zb, acc, sem):
        cid = lax.axis_index("c")
        sid = lax.axis_index("s")
        ibase = sid * (_SRW // _J)
        ebase = sid * _SRW
        segbase = cid * _HSEG
        pltpu.sync_copy(z_h, zb)
        z0 = sid * (_ACCR // _NS)
        for off, ln in ((0, _J), (_J, _J), (2 * _J, _ACCR // _NS - 2 * _J)):
            pltpu.sync_copy(zb.at[pl.ds(0, ln)], acc.at[pl.ds(z0 + off, ln)])
        plsc.subcore_barrier()

        def body(g, carry):
            pltpu.sync_copy(destI.at[pl.ds(ibase + g * _SKJ, _SKJ)], idx_v)
            for r in range(_SKJ):
                for q in range(_J // 16):
                    v = idx_v[r, pl.ds(q * 16, 16)]
                    l = v - segbase
                    ok = (l >= 0) & (l < _HSEG)
                    dump = _HSEG + (v & (_J - 1))
                    lidx_v[r, pl.ds(q * 16, 16)] = jnp.where(ok, l, dump)
            for h in range(2):
                pltpu.sync_copy(
                    er_h.at[pl.ds(ebase + g * _SSTG + h * (_SSTG // 2),
                                  _SSTG // 2)], rows_v)
                for j in range(_SKJ // 2):
                    pltpu.sync_copy(rows_v.at[pl.ds(j * _J, _J)],
                                    acc.at[lidx_v.at[h * (_SKJ // 2) + j]],
                                    add=True)
            return carry
        lax.fori_loop(0, _SNST, body, 0)
        plsc.subcore_barrier()

        w0 = sid * (_HSEG // _NS)
        for off, ln in ((0, _J), (_J, _J), (2 * _J, _HSEG // _NS - 2 * _J)):
            pltpu.sync_copy(acc.at[pl.ds(w0 + off, ln)], zb.at[pl.ds(0, ln)])
            pltpu.sync_copy(zb.at[pl.ds(0, ln)],
                            out.at[pl.ds(segbase + w0 + off, ln)])

    return k(er, dest2d, zrows)


def _row(b):
    return b.reshape(1, D)


def kernel(n, edge_index, q_0, params):
    src = edge_index[0]
    dest = edge_index[1]

    enc_node_w = [(W, _row(b)) for W, b in params["enc_node"]]

    (V1, c1), (V2, c2), (V3, c3) = params["enc_edge"]
    V1p = jnp.zeros((8, D), _F32).at[0:4].set(V1)
    enc_edge_w = (V1p, _row(c1), V2, _row(c2), V3, _row(c3))

    edge_w = []
    for (W1, b1), (W2, b2), (W3, b3) in params["edge_mlps"]:
        edge_w.append((W1, _row(b1), W2, _row(b2), W3, _row(b3)))
    node_w = []
    for (W1, b1), (W2, b2), (W3, b3) in params["node_mlps"]:
        node_w.append((W1, _row(b1), W2, _row(b2), W3, _row(b3)))

    (D1, e1), (D2, e2), (D3, e3) = params["dec"]
    D3p = jnp.zeros((D, D), _F32).at[:, 0:3].set(D3)
    e3p = jnp.zeros((1, D), _F32).at[:, 0:3].set(e3.reshape(1, 3))
    dec_w = (D1, _row(e1), D2, _row(e2), D3p, e3p)

    q0p = jnp.zeros((N, D), _F32).at[:, 0:3].set(q_0)

    pad = E2 - E
    src_p = jnp.concatenate(
        [src, jnp.zeros((pad,), jnp.int32)]).reshape(E2 // _GCH, _GCH)
    dest_p = jnp.concatenate(
        [dest, jnp.zeros((pad,), jnp.int32)]).reshape(E2 // _GCH, _GCH)
    dscat = jnp.concatenate([dest, jnp.full((pad,), N, jnp.int32)])
    dest2d = dscat.reshape(E2 // _J, _J)
    zrows = jnp.zeros((_J, D), _F32)

    x0 = _enc_node(n, enc_node_w)

    qs, qd, gs, gd = _gather4_call(q0p, x0, x0, src_p, dest_p)

    ea1, er1 = _edge_pass1(qs, qd, gs, gd, enc_edge_w, edge_w[0])

    agg = _scatter_call(er1, dest2d, zrows)
    x1 = _node_pass(x0, agg, node_w[0])

    gs2, gd2 = _gather2_call(x1, x1, src_p, dest_p)
    er2 = _edge_pass2(ea1, gs2, gd2, edge_w[1])

    agg = _scatter_call(er2, dest2d, zrows)
    out = _node_dec(x1, agg, node_w[1], dec_w)
    return out[:, 0:3]
